# baseline (speedup 1.0000x reference)
import jax
import jax.numpy as jnp
from jax.experimental import pallas as pl

N_TOK = 32768
EMB = 64
USER = 16
EXPERTS = 64

BLK = 8192


def _gate_kernel(h_ref, u_ref, wg_ref, bg_ref, wb_ref, bb_ref, wl_ref,
                 blt_ref, out_ref):
    h = h_ref[...]
    u = u_ref[...]
    dn = (((1,), (1,)), ((), ()))
    wgb = jnp.concatenate([wg_ref[...], wb_ref[...]], axis=0)
    bias = jnp.concatenate([1.0 + bg_ref[...], bb_ref[...]], axis=1)
    gb = jax.lax.dot_general(u, wgb, dn,
                             preferred_element_type=jnp.float32) + bias
    h_t = h * gb[:, :EMB] + gb[:, EMB:]
    lt = jax.lax.dot_general(wl_ref[...], h_t, dn,
                             preferred_element_type=jnp.float32) + blt_ref[...]

    rows = jax.lax.broadcasted_iota(jnp.int32, lt.shape, 0)
    m1 = jnp.max(lt, axis=0, keepdims=True)
    i1 = jnp.min(jnp.where(lt == m1, rows, EXPERTS), axis=0, keepdims=True)
    rest = jnp.where(rows == i1, -jnp.inf, lt)
    m2 = jnp.max(rest, axis=0, keepdims=True)
    i2 = jnp.min(jnp.where(rest == m2, rows, EXPERTS), axis=0, keepdims=True)
    s1 = 1.0 / (1.0 + jnp.exp(m2 - m1))

    i1c = i1.T
    i2c = i2.T
    s1c = s1.T
    cols = jax.lax.broadcasted_iota(jnp.int32, (h.shape[0], EXPERTS), 1)
    out_ref[...] = (jnp.where(cols == i1c, s1c, 0.0) +
                    jnp.where(cols == i2c, 1.0 - s1c, 0.0))


@jax.jit
def _run(h, u, wg, bg2, wb, bb2, wl, blt):
    grid = (N_TOK // BLK,)
    tok_spec = lambda width: pl.BlockSpec((BLK, width), lambda i: (i, 0))
    full = lambda a: pl.BlockSpec(a.shape, lambda i: (0,) * a.ndim)
    return pl.pallas_call(
        _gate_kernel,
        grid=grid,
        in_specs=[
            tok_spec(EMB),
            tok_spec(USER),
            full(wg), full(bg2), full(wb), full(bb2), full(wl), full(blt),
        ],
        out_specs=tok_spec(EXPERTS),
        out_shape=jax.ShapeDtypeStruct((N_TOK, EXPERTS), jnp.float32),
    )(h, u, wg, bg2, wb, bb2, wl, blt)


def kernel(h, u, Wg, bg, Wb, bb, Wl, bl):
    return _run(h, u, Wg, bg.reshape(1, EMB), Wb, bb.reshape(1, EMB),
                Wl, bl.reshape(EXPERTS, 1))


# speedup vs baseline: 1.3405x; 1.3405x over previous
import jax
import jax.numpy as jnp
from jax.experimental import pallas as pl

N_TOK = 32768
EMB = 64
USER = 16
EXPERTS = 64

BLK = 8192


def _gate_kernel(h_ref, u_ref, wg_ref, bg_ref, wb_ref, bb_ref, wl_ref,
                 blt_ref, out_ref):
    h = h_ref[...]
    u = u_ref[...]
    dn = (((1,), (1,)), ((), ()))
    wl = wl_ref[...]
    gamma = jax.lax.dot_general(u, wg_ref[...], dn,
                                preferred_element_type=jnp.float32)
    hm = h * (gamma + (1.0 + bg_ref[...]))
    cw = jax.lax.dot_general(wl, wb_ref[...],
                             (((1,), (0,)), ((), ())),
                             preferred_element_type=jnp.float32)
    lt = (jax.lax.dot_general(wl, hm, dn,
                              preferred_element_type=jnp.float32) +
          jax.lax.dot_general(cw, u, dn,
                              preferred_element_type=jnp.float32) +
          blt_ref[...])

    rows = jax.lax.broadcasted_iota(jnp.int32, lt.shape, 0)
    m1 = jnp.max(lt, axis=0, keepdims=True)
    i1 = jnp.min(jnp.where(lt == m1, rows, EXPERTS), axis=0, keepdims=True)
    sel1 = rows == i1
    rest = jnp.where(sel1, -jnp.inf, lt)
    m2 = jnp.max(rest, axis=0, keepdims=True)
    i2 = jnp.min(jnp.where(rest == m2, rows, EXPERTS), axis=0, keepdims=True)

    s1 = 1.0 / (1.0 + jnp.exp(m2 - m1))
    out_t = jnp.where(sel1, s1, 0.0) + jnp.where(rows == i2, 1.0 - s1, 0.0)
    out_ref[...] = out_t.T


@jax.jit
def _run(h, u, wg, bg2, wb, bb2, wl, blt):
    grid = (N_TOK // BLK,)
    tok_spec = lambda width: pl.BlockSpec((BLK, width), lambda i: (i, 0))
    full = lambda a: pl.BlockSpec(a.shape, lambda i: (0,) * a.ndim)
    return pl.pallas_call(
        _gate_kernel,
        grid=grid,
        in_specs=[
            tok_spec(EMB),
            tok_spec(USER),
            full(wg), full(bg2), full(wb), full(bb2), full(wl), full(blt),
        ],
        out_specs=tok_spec(EXPERTS),
        out_shape=jax.ShapeDtypeStruct((N_TOK, EXPERTS), jnp.float32),
    )(h, u, wg, bg2, wb, bb2, wl, blt)


def kernel(h, u, Wg, bg, Wb, bb, Wl, bl):
    dvec = (Wl @ bb + bl).reshape(EXPERTS, 1)
    return _run(h, u, Wg, bg.reshape(1, EMB), Wb, bb.reshape(1, EMB),
                Wl, dvec)


# speedup vs baseline: 1.3771x; 1.0273x over previous
import jax
import jax.numpy as jnp
from jax.experimental import pallas as pl

N_TOK = 32768
EMB = 64
USER = 16
EXPERTS = 64

BLK = 8192


def _gate_kernel(h_ref, u_ref, wg_ref, bg_ref, wb_ref, bb_ref, wl_ref,
                 blt_ref, out_ref):
    h = h_ref[...]
    u = u_ref[...]
    dn = (((1,), (1,)), ((), ()))
    wgb = jnp.concatenate([wg_ref[...], wb_ref[...]], axis=0)
    bias = jnp.concatenate([1.0 + bg_ref[...], bb_ref[...]], axis=1)
    gb = jax.lax.dot_general(u, wgb, dn,
                             preferred_element_type=jnp.float32) + bias
    h_t = h * gb[:, :EMB] + gb[:, EMB:]
    lt = jax.lax.dot_general(wl_ref[...], h_t, dn,
                             preferred_element_type=jnp.float32) + blt_ref[...]

    rows = jax.lax.broadcasted_iota(jnp.int32, lt.shape, 0)
    m1 = jnp.max(lt, axis=0, keepdims=True)
    i1 = jnp.argmax(lt, axis=0, keepdims=True).astype(jnp.int32)
    sel1 = rows == i1
    rest = jnp.where(sel1, -jnp.inf, lt)
    m2 = jnp.max(rest, axis=0, keepdims=True)
    i2 = jnp.argmax(rest, axis=0, keepdims=True).astype(jnp.int32)

    s1 = 1.0 / (1.0 + jnp.exp(m2 - m1))
    out_t = jnp.where(sel1, s1, 0.0) + jnp.where(rows == i2, 1.0 - s1, 0.0)
    out_ref[...] = out_t.T


@jax.jit
def _run(h, u, wg, bg2, wb, bb2, wl, blt):
    grid = (N_TOK // BLK,)
    tok_spec = lambda width: pl.BlockSpec((BLK, width), lambda i: (i, 0))
    full = lambda a: pl.BlockSpec(a.shape, lambda i: (0,) * a.ndim)
    return pl.pallas_call(
        _gate_kernel,
        grid=grid,
        in_specs=[
            tok_spec(EMB),
            tok_spec(USER),
            full(wg), full(bg2), full(wb), full(bb2), full(wl), full(blt),
        ],
        out_specs=tok_spec(EXPERTS),
        out_shape=jax.ShapeDtypeStruct((N_TOK, EXPERTS), jnp.float32),
    )(h, u, wg, bg2, wb, bb2, wl, blt)


def kernel(h, u, Wg, bg, Wb, bb, Wl, bl):
    return _run(h, u, Wg, bg.reshape(1, EMB), Wb, bb.reshape(1, EMB),
                Wl, bl.reshape(EXPERTS, 1))
